# baseline (speedup 1.0000x reference)
import functools

import jax
import jax.numpy as jnp
from jax import lax
from jax.experimental import pallas as pl
from jax.experimental.pallas import tpu as pltpu
from jax.experimental.pallas import tpu_sc as plsc

_NC = 2
_NS = 16
_NW = _NC * _NS
_LANES = 16


@functools.lru_cache(maxsize=None)
def _build(n, g):
    rpw = n // _NW
    chunk = min(rpw, 2048)
    nchunks = rpw // chunk
    assert rpw * _NW == n and nchunks * chunk == rpw

    eps = jnp.float32(1e-10)
    hi = jnp.float32(1.0 - 1e-10)
    step = jnp.float32(1.0 / (g - 1.0))
    gmax = jnp.int32(g - 1)

    mesh = plsc.VectorSubcoreMesh(core_axis_name="c", subcore_axis_name="s")

    @functools.partial(
        pl.kernel,
        mesh=mesh,
        compiler_params=pltpu.CompilerParams(
            needs_layout_passes=False, use_tc_tiling_on_sc=False),
        out_type=jax.ShapeDtypeStruct((n,), jnp.float32),
        scratch_types=[
            pltpu.VMEM((g, g), jnp.float32),
            pltpu.VMEM((chunk, 2), jnp.float32),
            pltpu.VMEM((chunk,), jnp.float32),
        ],
    )
    def run(obs_hbm, grid_hbm, out_hbm, grid_v, obs_v, out_v):
        wid = lax.axis_index("s") * _NC + lax.axis_index("c")
        base = wid * rpw
        pltpu.sync_copy(grid_hbm, grid_v)

        def chunk_body(c, _):
            row0 = base + c * chunk
            pltpu.sync_copy(obs_hbm.at[pl.ds(row0, chunk)], obs_v)

            def vec_body(k, _):
                r = k * _LANES + lax.broadcasted_iota(jnp.int32, (_LANES,), 0)
                zero = jnp.zeros((_LANES,), jnp.int32)
                u = plsc.load_gather(obs_v, [r, zero])
                v = plsc.load_gather(obs_v, [r, zero + 1])
                pu = jnp.minimum(jnp.maximum(u, eps), hi) / step
                pv = jnp.minimum(jnp.maximum(v, eps), hi) / step
                i0u = pu.astype(jnp.int32)
                i0v = pv.astype(jnp.int32)
                du = pu - i0u.astype(jnp.float32)
                dv = pv - i0v.astype(jnp.float32)
                i1u = jnp.minimum(i0u + 1, gmax)
                i1v = jnp.minimum(i0v + 1, gmax)
                g00 = plsc.load_gather(grid_v, [i0u, i0v])
                g10 = plsc.load_gather(grid_v, [i1u, i0v])
                g01 = plsc.load_gather(grid_v, [i0u, i1v])
                g11 = plsc.load_gather(grid_v, [i1u, i1v])
                res = (g00
                       + (g10 - g00) * du
                       + (g01 - g00) * dv
                       + (g11 - g01 - g10 + g00) * (du * dv))
                res = jnp.maximum(res, jnp.float32(0.0))
                out_v[pl.ds(k * _LANES, _LANES)] = res
                return 0

            lax.fori_loop(0, chunk // _LANES, vec_body, 0)
            pltpu.sync_copy(out_v, out_hbm.at[pl.ds(row0, chunk)])
            return 0

        lax.fori_loop(0, nchunks, chunk_body, 0)

    return run


def kernel(obs, pdf_grid):
    n = obs.shape[0]
    g = pdf_grid.shape[0]
    out = _build(n, g)(obs, pdf_grid)
    return out.reshape(n, 1)


# speedup vs baseline: 25.4331x; 25.4331x over previous
import functools

import jax
import jax.numpy as jnp
from jax import lax
from jax.experimental import pallas as pl
from jax.experimental.pallas import tpu as pltpu
from jax.experimental.pallas import tpu_sc as plsc

_NC = 2
_NS = 16
_NW = _NC * _NS
_LANES = 16


@functools.lru_cache(maxsize=None)
def _build(n, g):
    rpw = n // _NW
    chunk = min(rpw, 2048)
    nchunks = rpw // chunk
    assert rpw * _NW == n and nchunks * chunk == rpw

    eps = jnp.float32(1e-10)
    hi = jnp.float32(1.0 - 1e-10)
    inv_step = jnp.float32(g - 1.0)
    gmax = jnp.int32(g - 1)

    mesh = plsc.VectorSubcoreMesh(core_axis_name="c", subcore_axis_name="s")

    @functools.partial(
        pl.kernel,
        mesh=mesh,
        compiler_params=pltpu.CompilerParams(
            needs_layout_passes=False, use_tc_tiling_on_sc=False),
        out_type=jax.ShapeDtypeStruct((n,), jnp.float32),
        scratch_types=[
            pltpu.VMEM((g, g), jnp.float32),
            pltpu.VMEM((chunk,), jnp.float32),
            pltpu.VMEM((chunk,), jnp.float32),
            pltpu.VMEM((chunk,), jnp.float32),
        ],
    )
    def run(obs_hbm, grid_hbm, out_hbm, grid_v, u_v, v_v, out_v):
        wid = lax.axis_index("s") * _NC + lax.axis_index("c")
        base = wid * rpw
        pltpu.sync_copy(grid_hbm, grid_v)

        def chunk_body(c, _):
            row0 = base + c * chunk
            pltpu.sync_copy(obs_hbm.at[0, pl.ds(row0, chunk)], u_v)
            pltpu.sync_copy(obs_hbm.at[1, pl.ds(row0, chunk)], v_v)

            @plsc.parallel_loop(0, chunk, step=_LANES, unroll=8)
            def vec_body(i):
                u = u_v[pl.ds(i, _LANES)]
                v = v_v[pl.ds(i, _LANES)]
                pu = jnp.minimum(jnp.maximum(u, eps), hi) * inv_step
                pv = jnp.minimum(jnp.maximum(v, eps), hi) * inv_step
                i0u = pu.astype(jnp.int32)
                i0v = pv.astype(jnp.int32)
                du = pu - i0u.astype(jnp.float32)
                dv = pv - i0v.astype(jnp.float32)
                i1u = jnp.minimum(i0u + 1, gmax)
                i1v = jnp.minimum(i0v + 1, gmax)
                g00 = plsc.load_gather(grid_v, [i0u, i0v])
                g10 = plsc.load_gather(grid_v, [i1u, i0v])
                g01 = plsc.load_gather(grid_v, [i0u, i1v])
                g11 = plsc.load_gather(grid_v, [i1u, i1v])
                res = (g00
                       + (g10 - g00) * du
                       + (g01 - g00) * dv
                       + (g11 - g01 - g10 + g00) * (du * dv))
                res = jnp.maximum(res, jnp.float32(0.0))
                out_v[pl.ds(i, _LANES)] = res

            pltpu.sync_copy(out_v, out_hbm.at[pl.ds(row0, chunk)])
            return 0

        lax.fori_loop(0, nchunks, chunk_body, 0)

    return run


def kernel(obs, pdf_grid):
    n = obs.shape[0]
    g = pdf_grid.shape[0]
    out = _build(n, g)(obs.T, pdf_grid)
    return out.reshape(n, 1)
